# baseline (speedup 1.0000x reference)
import functools

import jax
import jax.numpy as jnp
from jax import lax
from jax.experimental import pallas as pl
from jax.experimental.pallas import tpu as pltpu
from jax.experimental.pallas import tpu_sc as plsc

N = 10000
NP = 10240
E = 320000
D = 128
H = 128
G = 64

NC = 2
NS = 16
NT = NC * NS
EPT = E // NT
K = 128
NCHUNK = 79
EPTP = NCHUNK * K
RPS = NP // NS
SINK = NP - 1

_mesh = plsc.VectorSubcoreMesh(core_axis_name="c", subcore_axis_name="s")


@functools.partial(
    pl.kernel,
    out_type=jax.ShapeDtypeStruct((NT, NP), jnp.float32),
    mesh=_mesh,
    compiler_params=pltpu.CompilerParams(needs_layout_passes=False),
    scratch_types=[
        pltpu.VMEM((NP,), jnp.float32),
        pltpu.VMEM((EPT,), jnp.int32),
    ],
)
def _sc_degree(dst_hbm, out_hbm, hist, didx):
    c = lax.axis_index("c")
    s = lax.axis_index("s")
    wid = c * NS + s

    def zbody(i, carry):
        hist[pl.ds(i * 16, 16)] = jnp.zeros((16,), jnp.float32)
        return carry

    lax.fori_loop(0, NP // 16, zbody, 0)
    pltpu.sync_copy(dst_hbm.at[pl.ds(wid * EPT, EPT)], didx)
    ones16 = jnp.ones((16,), jnp.float32)

    def body(i, carry):
        idx16 = didx[pl.ds(i * 16, 16)]
        plsc.addupdate_scatter(hist, [idx16], ones16)
        return carry

    lax.fori_loop(0, EPT // 16, body, 0)
    pltpu.sync_copy(hist, out_hbm.at[wid])


@functools.partial(
    pl.kernel,
    out_type=jax.ShapeDtypeStruct((NC, NP, H), jnp.float32),
    mesh=_mesh,
    scratch_types=[
        pltpu.VMEM_SHARED((NP, H), jnp.float32),
        pltpu.VMEM((NCHUNK, K), jnp.int32),
        pltpu.VMEM((K,), jnp.int32),
        pltpu.VMEM((K,), jnp.int32),
        pltpu.VMEM((K, H), jnp.float32),
        pltpu.VMEM((K, H), jnp.float32),
        pltpu.SemaphoreType.DMA,
        pltpu.SemaphoreType.DMA,
        pltpu.SemaphoreType.DMA,
        pltpu.SemaphoreType.DMA,
    ],
)
def _sc_scatter(g_hbm, src_hbm, dst_hbm, out_hbm, acc_sh, didx,
                sidx_a, sidx_b, rows_a, rows_b,
                gsem_a, gsem_b, lsem_a, lsem_b):
    c = lax.axis_index("c")
    s = lax.axis_index("s")
    wid = c * NS + s
    pltpu.sync_copy(g_hbm.at[pl.ds(s * RPS, RPS)],
                    acc_sh.at[pl.ds(s * RPS, RPS)])
    pltpu.sync_copy(dst_hbm.at[wid], didx)
    plsc.subcore_barrier()

    def iload(ch, sbuf, lsem):
        pltpu.async_copy(src_hbm.at[wid, ch, 0], sbuf, lsem)

    def iload_wait(ch, sbuf, lsem):
        pltpu.make_async_copy(src_hbm.at[wid, ch, 0], sbuf, lsem).wait()

    def gather(sbuf, rows, gsem):
        pltpu.async_copy(g_hbm.at[sbuf], rows, gsem)

    def gather_wait(sbuf, rows, gsem):
        pltpu.make_async_copy(g_hbm.at[sbuf], rows, gsem).wait()

    def scat(ch, rows):
        pltpu.sync_copy(rows, acc_sh.at[didx.at[ch]], add=True)

    iload(0, sidx_a, lsem_a)
    iload(1, sidx_b, lsem_b)
    iload_wait(0, sidx_a, lsem_a)
    gather(sidx_a, rows_a, gsem_a)

    def body(i, carry):
        a = 2 * i
        iload_wait(a + 1, sidx_b, lsem_b)
        gather(sidx_b, rows_b, gsem_b)
        gather_wait(sidx_a, rows_a, gsem_a)
        iload(a + 2, sidx_a, lsem_a)
        scat(a, rows_a)
        iload_wait(a + 2, sidx_a, lsem_a)
        gather(sidx_a, rows_a, gsem_a)
        gather_wait(sidx_b, rows_b, gsem_b)

        @pl.when(a + 3 < NCHUNK)
        def _():
            iload(a + 3, sidx_b, lsem_b)

        scat(a + 1, rows_b)
        return carry

    lax.fori_loop(0, (NCHUNK - 1) // 2, body, 0)
    gather_wait(sidx_a, rows_a, gsem_a)
    scat(NCHUNK - 1, rows_a)
    plsc.subcore_barrier()
    pltpu.sync_copy(acc_sh.at[pl.ds(s * RPS, RPS)],
                    out_hbm.at[c, pl.ds(s * RPS, RPS)])


BR = 640
NB = NP // BR


def _tc_stage1_body(deg_ref, x_ref, w1_ref, g1_ref, dinv_ref):
    ones_nt = jnp.ones((NT, 1), dtype=jnp.float32)
    deg = lax.dot_general(deg_ref[...], ones_nt, (((0,), (0,)), ((), ())),
                          preferred_element_type=jnp.float32) + 1.0
    dinv = lax.rsqrt(jnp.maximum(deg, 1e-12))
    h = jnp.dot(x_ref[...], w1_ref[...], preferred_element_type=jnp.float32)
    g1_ref[...] = h * dinv
    dinv_ref[...] = dinv


def _tc_stage1(deg, x, W1):
    return pl.pallas_call(
        _tc_stage1_body,
        grid=(NB,),
        in_specs=[
            pl.BlockSpec((NT, BR), lambda i: (0, i)),
            pl.BlockSpec((BR, D), lambda i: (i, 0)),
            pl.BlockSpec((D, H), lambda i: (0, 0)),
        ],
        out_specs=[
            pl.BlockSpec((BR, H), lambda i: (i, 0)),
            pl.BlockSpec((BR, 1), lambda i: (i, 0)),
        ],
        out_shape=[
            jax.ShapeDtypeStruct((NP, H), jnp.float32),
            jax.ShapeDtypeStruct((NP, 1), jnp.float32),
        ],
    )(deg, x, W1)


def _tc_stage2_body(acc_ref, g1_ref, dinv_ref, b1_ref, w2_ref, g2_ref):
    s1 = acc_ref[0] + acc_ref[1] - g1_ref[...]
    h1 = jnp.maximum(dinv_ref[...] * s1 + b1_ref[...], 0.0)
    h = jnp.dot(h1, w2_ref[...], preferred_element_type=jnp.float32)
    g2_ref[...] = h * dinv_ref[...]


def _tc_stage2(acc, g1, dinv, b1, W2):
    return pl.pallas_call(
        _tc_stage2_body,
        grid=(NB,),
        in_specs=[
            pl.BlockSpec((NC, BR, H), lambda i: (0, i, 0)),
            pl.BlockSpec((BR, H), lambda i: (i, 0)),
            pl.BlockSpec((BR, 1), lambda i: (i, 0)),
            pl.BlockSpec((1, H), lambda i: (0, 0)),
            pl.BlockSpec((H, H), lambda i: (0, 0)),
        ],
        out_specs=pl.BlockSpec((BR, H), lambda i: (i, 0)),
        out_shape=jax.ShapeDtypeStruct((NP, H), jnp.float32),
    )(acc, g1, dinv, b1, W2)


def _tc_stage3_body(acc_ref, g2_ref, dinv_ref, b2_ref, batch_ref, wh_ref,
                    bh_ref, out_ref, emb_acc, cnt_acc):
    i = pl.program_id(0)

    @pl.when(i == 0)
    def _():
        emb_acc[...] = jnp.zeros_like(emb_acc)
        cnt_acc[...] = jnp.zeros_like(cnt_acc)

    s2 = acc_ref[0] + acc_ref[1] - g2_ref[...]
    h2 = jnp.maximum(dinv_ref[...] * s2 + b2_ref[...], 0.0)
    seg = jax.lax.broadcasted_iota(jnp.int32, (G, BR), 0)
    p = (seg == batch_ref[0, 0, :][None, :]).astype(jnp.float32)
    emb_acc[...] += jnp.dot(p, h2, preferred_element_type=jnp.float32)
    cnt_acc[...] += jnp.sum(p, axis=1, keepdims=True)

    @pl.when(i == NB - 1)
    def _():
        emb = emb_acc[...] / jnp.maximum(cnt_acc[...], 1.0)
        out_ref[...] = (jnp.dot(emb, wh_ref[...],
                                preferred_element_type=jnp.float32)
                        + bh_ref[...])


def _tc_stage3(acc, g2, dinv, b2, batch_r, Whp, bhp):
    return pl.pallas_call(
        _tc_stage3_body,
        grid=(NB,),
        in_specs=[
            pl.BlockSpec((NC, BR, H), lambda i: (0, i, 0)),
            pl.BlockSpec((BR, H), lambda i: (i, 0)),
            pl.BlockSpec((BR, 1), lambda i: (i, 0)),
            pl.BlockSpec((1, H), lambda i: (0, 0)),
            pl.BlockSpec((1, 1, BR), lambda i: (i, 0, 0)),
            pl.BlockSpec((H, 128), lambda i: (0, 0)),
            pl.BlockSpec((1, 128), lambda i: (0, 0)),
        ],
        out_specs=pl.BlockSpec((G, 128), lambda i: (0, 0)),
        out_shape=jax.ShapeDtypeStruct((G, 128), jnp.float32),
        scratch_shapes=[
            pltpu.VMEM((G, 128), jnp.float32),
            pltpu.VMEM((G, 1), jnp.float32),
        ],
    )(acc, g2, dinv, b2, batch_r, Whp, bhp)


def kernel(x, edge_index, batch, W1, b1, W2, b2, Wh, bh):
    src = edge_index[0]
    dst = edge_index[1]
    x = jnp.pad(x, ((0, NP - N), (0, 0)))

    npad = EPTP - EPT
    src4 = jnp.concatenate(
        [src.reshape(NT, EPT), jnp.zeros((NT, npad), jnp.int32)],
        axis=1).reshape(NT, NCHUNK, 1, K)
    dst3 = jnp.concatenate(
        [dst.reshape(NT, EPT), jnp.full((NT, npad), SINK, jnp.int32)],
        axis=1).reshape(NT, NCHUNK, K)

    deg = _sc_degree(dst)
    g1, dinv = _tc_stage1(deg, x, W1)
    acc1 = _sc_scatter(g1, src4, dst3)
    g2 = _tc_stage2(acc1, g1, dinv, b1.reshape(1, H), W2)
    acc2 = _sc_scatter(g2, src4, dst3)

    batch_r = jnp.pad(batch, (0, NP - N), constant_values=G).reshape(NB, 1, BR)
    Whp = jnp.zeros((H, 128), dtype=jnp.float32).at[:, :2].set(Wh)
    bhp = jnp.zeros((1, 128), dtype=jnp.float32).at[0, :2].set(bh)
    out_pad = _tc_stage3(acc2, g2, dinv, b2.reshape(1, H), batch_r, Whp, bhp)
    return out_pad[:, :2]


# speedup vs baseline: 1.7540x; 1.7540x over previous
import functools

import jax
import jax.numpy as jnp
from jax import lax
from jax.experimental import pallas as pl
from jax.experimental.pallas import tpu as pltpu
from jax.experimental.pallas import tpu_sc as plsc

N = 10000
NP = 10240
E = 320000
D = 128
H = 128
G = 64

NC = 2
NS = 16
NT = NC * NS
EPT = E // NT
K = 80
NCHUNK = EPT // K
RPS = NP // NS
DW = 16

_mesh = plsc.VectorSubcoreMesh(core_axis_name="c", subcore_axis_name="s")


@functools.partial(
    pl.kernel,
    out_type=jax.ShapeDtypeStruct((NT, NP), jnp.float32),
    mesh=_mesh,
    compiler_params=pltpu.CompilerParams(needs_layout_passes=False),
    scratch_types=[
        pltpu.VMEM((NP,), jnp.float32),
        pltpu.VMEM((EPT,), jnp.int32),
    ],
)
def _sc_degree(dst_hbm, out_hbm, hist, didx):
    c = lax.axis_index("c")
    s = lax.axis_index("s")
    wid = c * NS + s

    def zbody(i, carry):
        hist[pl.ds(i * 16, 16)] = jnp.zeros((16,), jnp.float32)
        return carry

    lax.fori_loop(0, NP // 16, zbody, 0)
    pltpu.sync_copy(dst_hbm.at[pl.ds(wid * EPT, EPT)], didx)
    ones16 = jnp.ones((16,), jnp.float32)

    def body(i, carry):
        idx16 = didx[pl.ds(i * 16, 16)]
        plsc.addupdate_scatter(hist, [idx16], ones16)
        return carry

    lax.fori_loop(0, EPT // 16, body, 0)
    pltpu.sync_copy(hist, out_hbm.at[wid])


@functools.partial(
    pl.kernel,
    out_type=jax.ShapeDtypeStruct((NC, NP, H), jnp.float32),
    mesh=_mesh,
    scratch_types=[
        pltpu.VMEM_SHARED((NP, H), jnp.float32),
        pltpu.VMEM((EPT,), jnp.int32),
        pltpu.VMEM((NCHUNK, K), jnp.int32),
        pltpu.VMEM((K, H), jnp.float32),
        pltpu.VMEM((K, H), jnp.float32),
        pltpu.SemaphoreType.DMA,
        pltpu.SemaphoreType.DMA,
        pltpu.SemaphoreType.DMA,
    ],
)
def _sc_scatter(g_hbm, src_hbm, dst_hbm, out_hbm, acc_sh, sidx, didx,
                rows_a, rows_b, sem_a, sem_b, sem_p):
    c = lax.axis_index("c")
    s = lax.axis_index("s")
    wid = c * NS + s
    init_desc = pltpu.async_copy(g_hbm.at[pl.ds(s * RPS, RPS)],
                                 acc_sh.at[pl.ds(s * RPS, RPS)], sem_p)
    pltpu.async_copy(src_hbm.at[pl.ds(wid * EPT, EPT)], sidx, sem_a)
    pltpu.async_copy(dst_hbm.at[wid], didx, sem_b)
    init_desc.wait()
    pltpu.make_async_copy(src_hbm.at[pl.ds(wid * EPT, EPT)], sidx,
                          sem_a).wait()
    pltpu.make_async_copy(dst_hbm.at[wid], didx, sem_b).wait()
    plsc.subcore_barrier()

    def gather(ch, rows, sem):
        return pltpu.async_copy(g_hbm.at[sidx.at[pl.ds(ch * K, K)]], rows, sem)

    def gather_wait(ch, rows, sem):
        pltpu.make_async_copy(g_hbm.at[sidx.at[pl.ds(ch * K, K)]], rows,
                              sem).wait()

    def scat(ch, rows):
        pltpu.sync_copy(rows, acc_sh.at[didx.at[ch]], add=True)

    gather(0, rows_a, sem_a)

    def body(i, carry):
        a = 2 * i
        gather(a + 1, rows_b, sem_b)
        gather_wait(a, rows_a, sem_a)
        scat(a, rows_a)
        gather(a + 2, rows_a, sem_a)
        gather_wait(a + 1, rows_b, sem_b)
        scat(a + 1, rows_b)
        return carry

    lax.fori_loop(0, (NCHUNK - 1) // 2, body, 0)
    gather_wait(NCHUNK - 1, rows_a, sem_a)
    scat(NCHUNK - 1, rows_a)
    plsc.subcore_barrier()
    pltpu.sync_copy(acc_sh.at[pl.ds(s * RPS, RPS)],
                    out_hbm.at[c, pl.ds(s * RPS, RPS)])


BR = 640
NB = NP // BR


def _tc_stage1_body(deg_ref, x_ref, w1_ref, g1_ref, dinv_ref):
    ones_nt = jnp.ones((NT, 1), dtype=jnp.float32)
    deg = lax.dot_general(deg_ref[...], ones_nt, (((0,), (0,)), ((), ())),
                          preferred_element_type=jnp.float32) + 1.0
    dinv = lax.rsqrt(jnp.maximum(deg, 1e-12))
    h = jnp.dot(x_ref[...], w1_ref[...], preferred_element_type=jnp.float32)
    g1_ref[...] = h * dinv
    dinv_ref[...] = dinv


def _tc_stage1(deg, x, W1):
    return pl.pallas_call(
        _tc_stage1_body,
        grid=(NB,),
        in_specs=[
            pl.BlockSpec((NT, BR), lambda i: (0, i)),
            pl.BlockSpec((BR, D), lambda i: (i, 0)),
            pl.BlockSpec((D, H), lambda i: (0, 0)),
        ],
        out_specs=[
            pl.BlockSpec((BR, H), lambda i: (i, 0)),
            pl.BlockSpec((BR, 1), lambda i: (i, 0)),
        ],
        out_shape=[
            jax.ShapeDtypeStruct((NP, H), jnp.float32),
            jax.ShapeDtypeStruct((NP, 1), jnp.float32),
        ],
    )(deg, x, W1)


def _tc_stage2_body(acc_ref, g1_ref, dinv_ref, b1_ref, w2_ref, g2_ref):
    s1 = acc_ref[0] + acc_ref[1] - g1_ref[...]
    h1 = jnp.maximum(dinv_ref[...] * s1 + b1_ref[...], 0.0)
    h = jnp.dot(h1, w2_ref[...], preferred_element_type=jnp.float32)
    g2_ref[...] = h * dinv_ref[...]


def _tc_stage2(acc, g1, dinv, b1, W2):
    return pl.pallas_call(
        _tc_stage2_body,
        grid=(NB,),
        in_specs=[
            pl.BlockSpec((NC, BR, H), lambda i: (0, i, 0)),
            pl.BlockSpec((BR, H), lambda i: (i, 0)),
            pl.BlockSpec((BR, 1), lambda i: (i, 0)),
            pl.BlockSpec((1, H), lambda i: (0, 0)),
            pl.BlockSpec((H, H), lambda i: (0, 0)),
        ],
        out_specs=pl.BlockSpec((BR, H), lambda i: (i, 0)),
        out_shape=jax.ShapeDtypeStruct((NP, H), jnp.float32),
    )(acc, g1, dinv, b1, W2)


def _tc_stage3_body(acc_ref, g2_ref, dinv_ref, b2_ref, batch_ref, wh_ref,
                    bh_ref, out_ref, emb_acc, cnt_acc):
    i = pl.program_id(0)

    @pl.when(i == 0)
    def _():
        emb_acc[...] = jnp.zeros_like(emb_acc)
        cnt_acc[...] = jnp.zeros_like(cnt_acc)

    s2 = acc_ref[0] + acc_ref[1] - g2_ref[...]
    h2 = jnp.maximum(dinv_ref[...] * s2 + b2_ref[...], 0.0)
    seg = jax.lax.broadcasted_iota(jnp.int32, (G, BR), 0)
    p = (seg == batch_ref[0, 0, :][None, :]).astype(jnp.float32)
    emb_acc[...] += jnp.dot(p, h2, preferred_element_type=jnp.float32)
    cnt_acc[...] += jnp.sum(p, axis=1, keepdims=True)

    @pl.when(i == NB - 1)
    def _():
        emb = emb_acc[...] / jnp.maximum(cnt_acc[...], 1.0)
        out_ref[...] = (jnp.dot(emb, wh_ref[...],
                                preferred_element_type=jnp.float32)
                        + bh_ref[...])


def _tc_stage3(acc, g2, dinv, b2, batch_r, Whp, bhp):
    return pl.pallas_call(
        _tc_stage3_body,
        grid=(NB,),
        in_specs=[
            pl.BlockSpec((NC, BR, H), lambda i: (0, i, 0)),
            pl.BlockSpec((BR, H), lambda i: (i, 0)),
            pl.BlockSpec((BR, 1), lambda i: (i, 0)),
            pl.BlockSpec((1, H), lambda i: (0, 0)),
            pl.BlockSpec((1, 1, BR), lambda i: (i, 0, 0)),
            pl.BlockSpec((H, 128), lambda i: (0, 0)),
            pl.BlockSpec((1, 128), lambda i: (0, 0)),
        ],
        out_specs=pl.BlockSpec((G, 128), lambda i: (0, 0)),
        out_shape=jax.ShapeDtypeStruct((G, 128), jnp.float32),
        scratch_shapes=[
            pltpu.VMEM((G, 128), jnp.float32),
            pltpu.VMEM((G, 1), jnp.float32),
        ],
    )(acc, g2, dinv, b2, batch_r, Whp, bhp)


def kernel(x, edge_index, batch, W1, b1, W2, b2, Wh, bh):
    src = edge_index[0]
    dst = edge_index[1]
    x = jnp.pad(x, ((0, NP - N), (0, 0)))

    dst3 = dst.reshape(NT, NCHUNK, K)

    deg = _sc_degree(dst)
    g1, dinv = _tc_stage1(deg, x, W1)
    acc1 = _sc_scatter(g1, src, dst3)
    g2 = _tc_stage2(acc1, g1, dinv, b1.reshape(1, H), W2)
    acc2 = _sc_scatter(g2, src, dst3)

    batch_r = jnp.pad(batch, (0, NP - N), constant_values=G).reshape(NB, 1, BR)
    Whp = jnp.zeros((H, 128), dtype=jnp.float32).at[:, :2].set(Wh)
    bhp = jnp.zeros((1, 128), dtype=jnp.float32).at[0, :2].set(bh)
    out_pad = _tc_stage3(acc2, g2, dinv, b2.reshape(1, H), batch_r, Whp, bhp)
    return out_pad[:, :2]
